# baseline (speedup 1.0000x reference)
import functools

import jax
import jax.numpy as jnp
from jax import lax
from jax.experimental import pallas as pl
from jax.experimental.pallas import tpu as pltpu
from jax.experimental.pallas import tpu_sc as plsc

N = 20000
MAX_OUT = 300
IOU_THRESH = 0.5
NEG_F = -1e30
L = 16
NW = 16
N_PAD = 20480
PER_W = N_PAD // NW
CHUNKS = PER_W // L
DUMMY = 3e9
BIG = 4e9


def _nms_body(x1_hbm, y1_hbm, x2_hbm, y2_hbm, s_hbm, out_hbm,
              x1_v, y1_v, x2_v, y2_v, ar_v, sw_v,
              rec_v, all_v, out_v, shared):
    c = lax.axis_index("c")
    s = lax.axis_index("s")
    base = s * PER_W

    pltpu.sync_copy(x1_hbm.at[pl.ds(base, PER_W)], x1_v)
    pltpu.sync_copy(y1_hbm.at[pl.ds(base, PER_W)], y1_v)
    pltpu.sync_copy(x2_hbm.at[pl.ds(base, PER_W)], x2_v)
    pltpu.sync_copy(y2_hbm.at[pl.ds(base, PER_W)], y2_v)
    pltpu.sync_copy(s_hbm.at[pl.ds(base, PER_W)], sw_v)

    def init_chunk(j, carry):
        sl = pl.ds(j * L, L)
        ar_v[sl] = (x2_v[sl] - x1_v[sl]) * (y2_v[sl] - y1_v[sl])
        return carry
    lax.fori_loop(0, CHUNKS, init_chunk, 0)

    iota = lax.iota(jnp.int32, L)
    iotaf = iota.astype(jnp.float32)
    zeros_i = jnp.zeros((L,), jnp.int32)
    neg_f = jnp.float32(NEG_F)
    half = jnp.float32(IOU_THRESH)
    eps = jnp.float32(1e-9)
    so0 = plsc.load_gather(sw_v, [zeros_i])[0]

    def outer(i, carry):
        bx1, by1, bx2, by2, bar = carry

        @plsc.parallel_loop(
            0, PER_W, L, unroll=8,
            carry=(jnp.full((L,), -3.4e38, jnp.float32),
                   jnp.full((L,), BIG, jnp.float32),
                   iotaf + jnp.float32(base)))
        def pass_result(j, pc):
            run_val, run_idx, idxv = pc
            sl = pl.ds(j, L)
            xx1 = jnp.maximum(x1_v[sl], bx1)
            yy1 = jnp.maximum(y1_v[sl], by1)
            xx2 = jnp.minimum(x2_v[sl], bx2)
            yy2 = jnp.minimum(y2_v[sl], by2)
            w = jnp.maximum(xx2 - xx1, jnp.float32(0.0))
            h = jnp.maximum(yy2 - yy1, jnp.float32(0.0))
            inter = w * h
            iou = inter / (ar_v[sl] + bar - inter + eps)
            ns = jnp.where(iou > half, neg_f, sw_v[sl])
            sw_v[sl] = ns
            gt = ns > run_val
            run_val = jnp.maximum(run_val, ns)
            run_idx = jnp.where(gt, idxv, run_idx)
            return run_val, run_idx, idxv + jnp.float32(L)

        run_val, run_idx, _ = pass_result

        m = jnp.max(run_val)
        li = jnp.min(jnp.where(run_val == m, run_idx, jnp.float32(BIG)))
        lov = zeros_i + (li.astype(jnp.int32) - base)

        rec = jnp.where(iota == 0, m, jnp.float32(0.0))
        rec = jnp.where(iota == 1, li, rec)
        for k, arr in ((2, x1_v), (3, y1_v), (4, x2_v), (5, y2_v)):
            rec = jnp.where(iota == k, plsc.load_gather(arr, [lov]), rec)
        rec = jnp.where(iota == 6, jnp.where(m == neg_f, so0, m), rec)
        rec_v[...] = rec

        p = (i & 1) * (NW * L)
        pltpu.sync_copy(rec_v, shared.at[pl.ds(p + s * L, L)])
        plsc.subcore_barrier()
        pltpu.sync_copy(shared.at[pl.ds(p, NW * L)], all_v)

        vals = plsc.load_gather(all_v, [iota * L])
        gm = jnp.max(vals)
        wl = jnp.min(jnp.where(vals == gm, iota, L))
        rec_win = plsc.load_gather(all_v, [wl * L + iota])
        nx1 = rec_win[2]
        ny1 = rec_win[3]
        nx2 = rec_win[4]
        ny2 = rec_win[5]
        nar = (nx2 - nx1) * (ny2 - ny1)

        orow = jnp.zeros((L,), jnp.float32)
        for k, val in enumerate((nx1, ny1, nx2, ny2, rec_win[6])):
            orow = jnp.where(iota == k, val, orow)
        out_v[i] = orow

        return nx1, ny1, nx2, ny2, nar

    dummy = jnp.float32(DUMMY)
    lax.fori_loop(0, MAX_OUT, outer,
                  (dummy, dummy, dummy, dummy, jnp.float32(0.0)))

    @pl.when((c == 0) & (s == 0))
    def _():
        pltpu.sync_copy(out_v, out_hbm)


@functools.partial(
    pl.kernel,
    out_type=jax.ShapeDtypeStruct((MAX_OUT, L), jnp.float32),
    mesh=plsc.VectorSubcoreMesh(core_axis_name="c", subcore_axis_name="s"),
    compiler_params=pltpu.CompilerParams(needs_layout_passes=False),
    scratch_types=[
        pltpu.VMEM((PER_W,), jnp.float32),
        pltpu.VMEM((PER_W,), jnp.float32),
        pltpu.VMEM((PER_W,), jnp.float32),
        pltpu.VMEM((PER_W,), jnp.float32),
        pltpu.VMEM((PER_W,), jnp.float32),
        pltpu.VMEM((PER_W,), jnp.float32),
        pltpu.VMEM((L,), jnp.float32),
        pltpu.VMEM((NW * L,), jnp.float32),
        pltpu.VMEM((MAX_OUT, L), jnp.float32),
        pltpu.VMEM_SHARED((2 * NW * L,), jnp.float32),
    ],
)
def _nms_sc(*refs):
    _nms_body(*refs)


@jax.jit
def kernel(boxes, scores):
    pad = N_PAD - N
    x1 = jnp.pad(boxes[:, 0], (0, pad))
    y1 = jnp.pad(boxes[:, 1], (0, pad))
    x2 = jnp.pad(boxes[:, 2], (0, pad))
    y2 = jnp.pad(boxes[:, 3], (0, pad))
    sp = jnp.pad(scores, (0, pad), constant_values=NEG_F)
    out = _nms_sc(x1, y1, x2, y2, sp)
    return out[:, :5]


# speedup vs baseline: 1.0736x; 1.0736x over previous
import functools

import jax
import jax.numpy as jnp
from jax import lax
from jax.experimental import pallas as pl
from jax.experimental.pallas import tpu as pltpu
from jax.experimental.pallas import tpu_sc as plsc

N = 20000
MAX_OUT = 300
IOU_THRESH = 0.5
NEG_F = -1e30
L = 16
NW = 16
N_PAD = 20480
PER_W = N_PAD // NW
CHUNKS = PER_W // L
DUMMY = 3e9
BIG = 4e9


def _nms_body(x1_hbm, y1_hbm, x2_hbm, y2_hbm, s_hbm, out_hbm,
              x1_v, y1_v, x2_v, y2_v, ar_v, sw_v,
              rec_v, all_v, out_v, shared):
    c = lax.axis_index("c")
    s = lax.axis_index("s")
    base = s * PER_W

    pltpu.sync_copy(x1_hbm.at[pl.ds(base, PER_W)], x1_v)
    pltpu.sync_copy(y1_hbm.at[pl.ds(base, PER_W)], y1_v)
    pltpu.sync_copy(x2_hbm.at[pl.ds(base, PER_W)], x2_v)
    pltpu.sync_copy(y2_hbm.at[pl.ds(base, PER_W)], y2_v)
    pltpu.sync_copy(s_hbm.at[pl.ds(base, PER_W)], sw_v)

    def init_chunk(j, carry):
        sl = pl.ds(j * L, L)
        ar_v[sl] = (x2_v[sl] - x1_v[sl]) * (y2_v[sl] - y1_v[sl])
        return carry
    lax.fori_loop(0, CHUNKS, init_chunk, 0)

    iota = lax.iota(jnp.int32, L)
    iotaf = iota.astype(jnp.float32)
    zeros_i = jnp.zeros((L,), jnp.int32)
    neg_f = jnp.float32(NEG_F)
    half = jnp.float32(IOU_THRESH)
    eps = jnp.float32(1e-9)
    so0 = plsc.load_gather(sw_v, [zeros_i])[0]

    def outer(i, carry):
        bx1, by1, bx2, by2, bar = carry

        @plsc.parallel_loop(
            0, PER_W, L, unroll=4,
            carry=(jnp.full((L,), -3.4e38, jnp.float32),
                   jnp.full((L,), BIG, jnp.float32),
                   iotaf + jnp.float32(base)))
        def pass_result(j, pc):
            run_val, run_idx, idxv = pc
            sl = pl.ds(j, L)
            xx1 = jnp.maximum(x1_v[sl], bx1)
            yy1 = jnp.maximum(y1_v[sl], by1)
            xx2 = jnp.minimum(x2_v[sl], bx2)
            yy2 = jnp.minimum(y2_v[sl], by2)
            w = jnp.maximum(xx2 - xx1, jnp.float32(0.0))
            h = jnp.maximum(yy2 - yy1, jnp.float32(0.0))
            inter = w * h
            sup = inter > half * (ar_v[sl] + bar - inter + eps)
            ns = jnp.where(sup, neg_f, sw_v[sl])
            sw_v[sl] = ns
            gt = ns > run_val
            run_val = jnp.maximum(run_val, ns)
            run_idx = jnp.where(gt, idxv, run_idx)
            return run_val, run_idx, idxv + jnp.float32(L)

        run_val, run_idx, _ = pass_result

        m = jnp.max(run_val)
        li = jnp.min(jnp.where(run_val == m, run_idx, jnp.float32(BIG)))
        lov = zeros_i + (li.astype(jnp.int32) - base)

        rec = jnp.where(iota == 0, m, jnp.float32(0.0))
        rec = jnp.where(iota == 1, li, rec)
        for k, arr in ((2, x1_v), (3, y1_v), (4, x2_v), (5, y2_v)):
            rec = jnp.where(iota == k, plsc.load_gather(arr, [lov]), rec)
        rec = jnp.where(iota == 6, jnp.where(m == neg_f, so0, m), rec)
        rec_v[...] = rec

        p = (i & 1) * (NW * L)
        pltpu.sync_copy(rec_v, shared.at[pl.ds(p + s * L, L)])
        plsc.subcore_barrier()
        pltpu.sync_copy(shared.at[pl.ds(p, NW * L)], all_v)

        vals = plsc.load_gather(all_v, [iota * L])
        gm = jnp.max(vals)
        wl = jnp.min(jnp.where(vals == gm, iota, L))
        rec_win = plsc.load_gather(all_v, [wl * L + iota])
        nx1 = rec_win[2]
        ny1 = rec_win[3]
        nx2 = rec_win[4]
        ny2 = rec_win[5]
        nar = (nx2 - nx1) * (ny2 - ny1)

        orow = jnp.zeros((L,), jnp.float32)
        for k, val in enumerate((nx1, ny1, nx2, ny2, rec_win[6])):
            orow = jnp.where(iota == k, val, orow)
        out_v[i] = orow

        return nx1, ny1, nx2, ny2, nar

    dummy = jnp.float32(DUMMY)
    lax.fori_loop(0, MAX_OUT, outer,
                  (dummy, dummy, dummy, dummy, jnp.float32(0.0)))

    @pl.when((c == 0) & (s == 0))
    def _():
        pltpu.sync_copy(out_v, out_hbm)


@functools.partial(
    pl.kernel,
    out_type=jax.ShapeDtypeStruct((MAX_OUT, L), jnp.float32),
    mesh=plsc.VectorSubcoreMesh(core_axis_name="c", subcore_axis_name="s"),
    compiler_params=pltpu.CompilerParams(needs_layout_passes=False),
    scratch_types=[
        pltpu.VMEM((PER_W,), jnp.float32),
        pltpu.VMEM((PER_W,), jnp.float32),
        pltpu.VMEM((PER_W,), jnp.float32),
        pltpu.VMEM((PER_W,), jnp.float32),
        pltpu.VMEM((PER_W,), jnp.float32),
        pltpu.VMEM((PER_W,), jnp.float32),
        pltpu.VMEM((L,), jnp.float32),
        pltpu.VMEM((NW * L,), jnp.float32),
        pltpu.VMEM((MAX_OUT, L), jnp.float32),
        pltpu.VMEM_SHARED((2 * NW * L,), jnp.float32),
    ],
)
def _nms_sc(*refs):
    _nms_body(*refs)


@jax.jit
def kernel(boxes, scores):
    pad = N_PAD - N
    x1 = jnp.pad(boxes[:, 0], (0, pad))
    y1 = jnp.pad(boxes[:, 1], (0, pad))
    x2 = jnp.pad(boxes[:, 2], (0, pad))
    y2 = jnp.pad(boxes[:, 3], (0, pad))
    sp = jnp.pad(scores, (0, pad), constant_values=NEG_F)
    out = _nms_sc(x1, y1, x2, y2, sp)
    return out[:, :5]
